# Initial kernel scaffold; baseline (speedup 1.0000x reference)
import jax
import jax.numpy as jnp
from jax.experimental import pallas as pl


def kernel(slice_matrices, qubit_embs, W1, b1, W2, b2):
    raise NotImplementedError("write your pallas kernel here")


# speedup vs baseline: 45.1274x; 45.1274x over previous
import functools

import jax
import jax.numpy as jnp
from jax import lax
from jax.experimental import pallas as pl
from jax.experimental.pallas import tpu as pltpu
from jax.experimental.pallas import tpu_sc as plsc

S, Q, E, D0, D1, D2 = 20, 500, 16000, 128, 128, 128
NP = 512
QUARTER = 128
NTASK = S * (NP // QUARTER)
NWORKER = 32
CHUNK = 2000
NCHUNK = E // CHUNK
GROUPS = CHUNK // 16

_sc_mesh = plsc.VectorSubcoreMesh(core_axis_name="c", subcore_axis_name="s")


@functools.partial(
    pl.kernel,
    out_type=jax.ShapeDtypeStruct((S, NP, NP), jnp.float32),
    mesh=_sc_mesh,
    scratch_types=[
        pltpu.VMEM((CHUNK,), jnp.int32),
        pltpu.VMEM((CHUNK,), jnp.int32),
        pltpu.VMEM((QUARTER, NP), jnp.float32),
    ],
    compiler_params=pltpu.CompilerParams(
        needs_layout_passes=False, use_tc_tiling_on_sc=False),
)
def _build_adjacency(sm_hbm, a_hbm, rows_v, cols_v, acc_v):
    wid = lax.axis_index("s") * 2 + lax.axis_index("c")
    zeros16 = jnp.zeros((16,), jnp.float32)
    ones16 = jnp.ones((16,), jnp.float32)

    for rnd in range((NTASK + NWORKER - 1) // NWORKER):
        task = rnd * NWORKER + wid

        @pl.when(task < NTASK)
        def _():
            sl = task % S
            base = (task // S) * QUARTER

            def zero_row(i, _):
                for j in range(NP // 16):
                    acc_v[i, pl.ds(j * 16, 16)] = zeros16
                return 0

            lax.fori_loop(0, QUARTER, zero_row, 0)

            for c in range(NCHUNK):
                pltpu.sync_copy(sm_hbm.at[sl, 0, pl.ds(c * CHUNK, CHUNK)], rows_v)
                pltpu.sync_copy(sm_hbm.at[sl, 1, pl.ds(c * CHUNK, CHUNK)], cols_v)

                def edge_group(g, _):
                    r16 = rows_v[pl.ds(g * 16, 16)]
                    c16 = cols_v[pl.ds(g * 16, 16)]
                    local = c16 - base
                    mask = (local >= 0) & (local < QUARTER)
                    local = jnp.where(mask, local, 0)
                    plsc.addupdate_scatter(acc_v, [local, r16], ones16, mask=mask)
                    return 0

                lax.fori_loop(0, GROUPS, edge_group, 0)

            pltpu.sync_copy(acc_v, a_hbm.at[sl, pl.ds(base, QUARTER), :])


def _gcn_body(a_ref, qe_ref, w1_ref, w2_ref, b1_ref, b2_ref, out_ref):
    a = a_ref[0]
    deg = jnp.sum(a, axis=1) + 1.0
    dinv = lax.rsqrt(deg)
    i0 = lax.broadcasted_iota(jnp.int32, (NP, NP), 0)
    i1 = lax.broadcasted_iota(jnp.int32, (NP, NP), 1)
    eye = jnp.where(i0 == i1, 1.0, 0.0)
    m = (a + eye) * dinv[:, None] * dinv[None, :]

    def mm(x, y):
        return jnp.dot(x, y, precision=lax.Precision.HIGHEST,
                       preferred_element_type=jnp.float32)

    h0 = mm(qe_ref[...], w1_ref[...])
    x1 = jnp.maximum(mm(m, h0) + b1_ref[...], 0.0)
    h2 = mm(x1, w2_ref[...])
    out_ref[0] = jnp.maximum(mm(m, h2) + b2_ref[...], 0.0)


def _gcn_tc(a, qe_pad, w1, w2, b1, b2):
    return pl.pallas_call(
        _gcn_body,
        grid=(S,),
        in_specs=[
            pl.BlockSpec((1, NP, NP), lambda s: (s, 0, 0)),
            pl.BlockSpec((NP, D0), lambda s: (0, 0)),
            pl.BlockSpec((D0, D1), lambda s: (0, 0)),
            pl.BlockSpec((D1, D2), lambda s: (0, 0)),
            pl.BlockSpec((1, D1), lambda s: (0, 0)),
            pl.BlockSpec((1, D2), lambda s: (0, 0)),
        ],
        out_specs=pl.BlockSpec((1, NP, D2), lambda s: (s, 0, 0)),
        out_shape=jax.ShapeDtypeStruct((S, NP, D2), jnp.float32),
    )(a, qe_pad, w1, w2, b1, b2)


def kernel(slice_matrices, qubit_embs, W1, b1, W2, b2):
    sm = slice_matrices.astype(jnp.int32)
    a = _build_adjacency(sm)
    qe_pad = jnp.zeros((NP, D0), jnp.float32).at[:Q].set(qubit_embs)
    out = _gcn_tc(a, qe_pad, W1, W2, b1.reshape(1, D1), b2.reshape(1, D2))
    return out[:, :Q, :].reshape(S * Q, D2)


# speedup vs baseline: 53.6516x; 1.1889x over previous
import functools

import jax
import jax.numpy as jnp
from jax import lax
from jax.experimental import pallas as pl
from jax.experimental.pallas import tpu as pltpu
from jax.experimental.pallas import tpu_sc as plsc

S, Q, E, D0, D1, D2 = 20, 500, 16000, 128, 128, 128
NP = 512
QUARTER = 128
NTASK = S * (NP // QUARTER)
NWORKER = 32
CHUNK = 2000
NCHUNK = E // CHUNK
GROUPS = CHUNK // 16

_sc_mesh = plsc.VectorSubcoreMesh(core_axis_name="c", subcore_axis_name="s")


@functools.partial(
    pl.kernel,
    out_type=jax.ShapeDtypeStruct((S, NP, NP), jnp.float32),
    mesh=_sc_mesh,
    scratch_types=[
        pltpu.VMEM((2, 2, CHUNK), jnp.int32),
        pltpu.VMEM((QUARTER, NP), jnp.float32),
        pltpu.SemaphoreType.DMA,
        pltpu.SemaphoreType.DMA,
    ],
    compiler_params=pltpu.CompilerParams(
        needs_layout_passes=False, use_tc_tiling_on_sc=False),
)
def _build_adjacency(sm_hbm, a_hbm, ebuf_v, acc_v, sem0, sem1):
    wid = lax.axis_index("s") * 2 + lax.axis_index("c")
    zeros16 = jnp.zeros((16,), jnp.float32)
    ones16 = jnp.ones((16,), jnp.float32)
    sems = (sem0, sem1)
    UNROLL = 5

    for rnd in range((NTASK + NWORKER - 1) // NWORKER):
        task = rnd * NWORKER + wid

        @pl.when(task < NTASK)
        def _():
            sl = task % S
            base = (task // S) * QUARTER

            copies = [None] * NCHUNK
            copies[0] = pltpu.async_copy(
                sm_hbm.at[sl, :, pl.ds(0, CHUNK)], ebuf_v.at[0], sems[0])

            def zero_row(i, _):
                for j in range(NP // 16):
                    acc_v[i, pl.ds(j * 16, 16)] = zeros16
                return 0

            lax.fori_loop(0, QUARTER, zero_row, 0)

            for c in range(NCHUNK):
                cur = c % 2
                if c + 1 < NCHUNK:
                    copies[c + 1] = pltpu.async_copy(
                        sm_hbm.at[sl, :, pl.ds((c + 1) * CHUNK, CHUNK)],
                        ebuf_v.at[1 - cur], sems[1 - cur])
                copies[c].wait()

                def edge_group(g, _):
                    for u in range(UNROLL):
                        off = g * (16 * UNROLL) + u * 16
                        r16 = ebuf_v[cur, 0, pl.ds(off, 16)]
                        c16 = ebuf_v[cur, 1, pl.ds(off, 16)]
                        local = c16 - base
                        mask = (local >= 0) & (local < QUARTER)
                        local = jnp.where(mask, local, 0)
                        plsc.addupdate_scatter(
                            acc_v, [local, r16], ones16, mask=mask)
                    return 0

                lax.fori_loop(0, GROUPS // UNROLL, edge_group, 0)

            pltpu.sync_copy(acc_v, a_hbm.at[sl, pl.ds(base, QUARTER), :])


def _gcn_body(a_ref, qe_ref, w1_ref, w2_ref, b1_ref, b2_ref, out_ref):
    a = a_ref[0]
    deg = jnp.sum(a, axis=1) + 1.0
    dinv = lax.rsqrt(deg)
    i0 = lax.broadcasted_iota(jnp.int32, (NP, NP), 0)
    i1 = lax.broadcasted_iota(jnp.int32, (NP, NP), 1)
    eye = jnp.where(i0 == i1, 1.0, 0.0)
    m = (a + eye) * dinv[:, None] * dinv[None, :]

    def mm(x, y):
        return jnp.dot(x, y, precision=lax.Precision.HIGHEST,
                       preferred_element_type=jnp.float32)

    h0 = mm(qe_ref[...], w1_ref[...])
    x1 = jnp.maximum(mm(m, h0) + b1_ref[...], 0.0)
    h2 = mm(x1, w2_ref[...])
    out_ref[0] = jnp.maximum(mm(m, h2) + b2_ref[...], 0.0)


def _gcn_tc(a, qe_pad, w1, w2, b1, b2):
    return pl.pallas_call(
        _gcn_body,
        grid=(S,),
        in_specs=[
            pl.BlockSpec((1, NP, NP), lambda s: (s, 0, 0)),
            pl.BlockSpec((NP, D0), lambda s: (0, 0)),
            pl.BlockSpec((D0, D1), lambda s: (0, 0)),
            pl.BlockSpec((D1, D2), lambda s: (0, 0)),
            pl.BlockSpec((1, D1), lambda s: (0, 0)),
            pl.BlockSpec((1, D2), lambda s: (0, 0)),
        ],
        out_specs=pl.BlockSpec((1, NP, D2), lambda s: (s, 0, 0)),
        out_shape=jax.ShapeDtypeStruct((S, NP, D2), jnp.float32),
    )(a, qe_pad, w1, w2, b1, b2)


def kernel(slice_matrices, qubit_embs, W1, b1, W2, b2):
    sm = slice_matrices.astype(jnp.int32)
    a = _build_adjacency(sm)
    qe_pad = jnp.zeros((NP, D0), jnp.float32).at[:Q].set(qubit_embs)
    out = _gcn_tc(a, qe_pad, W1, W2, b1.reshape(1, D1), b2.reshape(1, D2))
    return out[:, :Q, :].reshape(S * Q, D2)


# speedup vs baseline: 73.1920x; 1.3642x over previous
import functools

import jax
import jax.numpy as jnp
from jax import lax
from jax.experimental import pallas as pl
from jax.experimental.pallas import tpu as pltpu
from jax.experimental.pallas import tpu_sc as plsc

S, Q, E, D0, D1, D2 = 20, 500, 16000, 128, 128, 128
NP = 512
QUARTER = 128
NTASK = S * (NP // QUARTER)
NWORKER = 32
CHUNK = 2000
NCHUNK = E // CHUNK
GROUPS = CHUNK // 16

_sc_mesh = plsc.VectorSubcoreMesh(core_axis_name="c", subcore_axis_name="s")


@functools.partial(
    pl.kernel,
    out_type=jax.ShapeDtypeStruct((S, NP, NP), jnp.float32),
    mesh=_sc_mesh,
    scratch_types=[
        pltpu.VMEM((2, 2, CHUNK), jnp.int32),
        pltpu.VMEM((QUARTER, NP), jnp.float32),
        pltpu.SemaphoreType.DMA,
        pltpu.SemaphoreType.DMA,
    ],
    compiler_params=pltpu.CompilerParams(
        needs_layout_passes=False, use_tc_tiling_on_sc=False),
)
def _build_adjacency(sm_hbm, a_hbm, ebuf_v, acc_v, sem0, sem1):
    wid = lax.axis_index("s") * 2 + lax.axis_index("c")
    zeros16 = jnp.zeros((16,), jnp.float32)
    ones16 = jnp.ones((16,), jnp.float32)
    sems = (sem0, sem1)
    UNROLL = 5

    for rnd in range((NTASK + NWORKER - 1) // NWORKER):
        task = rnd * NWORKER + wid

        @pl.when(task < NTASK)
        def _():
            sl = task % S
            base = (task // S) * QUARTER

            copies = [None] * NCHUNK
            copies[0] = pltpu.async_copy(
                sm_hbm.at[sl, :, pl.ds(0, CHUNK)], ebuf_v.at[0], sems[0])

            def zero_row(i, _):
                for j in range(NP // 16):
                    acc_v[i, pl.ds(j * 16, 16)] = zeros16
                return 0

            lax.fori_loop(0, QUARTER, zero_row, 0)

            for c in range(NCHUNK):
                cur = c % 2
                if c + 1 < NCHUNK:
                    copies[c + 1] = pltpu.async_copy(
                        sm_hbm.at[sl, :, pl.ds((c + 1) * CHUNK, CHUNK)],
                        ebuf_v.at[1 - cur], sems[1 - cur])
                copies[c].wait()

                def edge_group(g, _):
                    for u in range(UNROLL):
                        off = g * (16 * UNROLL) + u * 16
                        r16 = ebuf_v[cur, 0, pl.ds(off, 16)]
                        c16 = ebuf_v[cur, 1, pl.ds(off, 16)]
                        local = c16 - base
                        mask = (local >= 0) & (local < QUARTER)
                        local = jnp.where(mask, local, 0)
                        plsc.addupdate_scatter(
                            acc_v, [local, r16], ones16, mask=mask)
                    return 0

                lax.fori_loop(0, GROUPS // UNROLL, edge_group, 0)

            pltpu.sync_copy(acc_v, a_hbm.at[sl, pl.ds(base, QUARTER), :])


def _split(x):
    hi = x.astype(jnp.bfloat16)
    lo = (x - hi.astype(jnp.float32)).astype(jnp.bfloat16)
    return hi, lo


def _mm3(xs, ys):
    xh, xl = xs
    yh, yl = ys
    def d(p, q):
        return jnp.dot(p, q, preferred_element_type=jnp.float32)
    return d(xh, yh) + d(xh, yl) + d(xl, yh)


def _gcn_body(a_ref, qe_ref, w1_ref, w2_ref, b1_ref, b2_ref, out_ref, h0_ref):
    @pl.when(pl.program_id(0) == 0)
    def _():
        h0_ref[...] = _mm3(_split(qe_ref[...]), _split(w1_ref[...]))

    a = a_ref[0]
    deg = jnp.sum(a, axis=1) + 1.0
    dinv = lax.rsqrt(deg)
    i0 = lax.broadcasted_iota(jnp.int32, (NP, NP), 0)
    i1 = lax.broadcasted_iota(jnp.int32, (NP, NP), 1)
    eye = jnp.where(i0 == i1, 1.0, 0.0)
    m = (a + eye) * dinv[:, None] * dinv[None, :]
    ms = _split(m)
    w2s = _split(w2_ref[...])

    x1 = jnp.maximum(_mm3(ms, _split(h0_ref[...])) + b1_ref[...], 0.0)
    h2 = _mm3(_split(x1), w2s)
    x2 = jnp.maximum(_mm3(ms, _split(h2)) + b2_ref[...], 0.0)
    out_ref[0] = x2[:Q]


def _gcn_tc(a, qe_pad, w1, w2, b1, b2):
    return pl.pallas_call(
        _gcn_body,
        grid=(S,),
        in_specs=[
            pl.BlockSpec((1, NP, NP), lambda s: (s, 0, 0)),
            pl.BlockSpec((NP, D0), lambda s: (0, 0)),
            pl.BlockSpec((D0, D1), lambda s: (0, 0)),
            pl.BlockSpec((D1, D2), lambda s: (0, 0)),
            pl.BlockSpec((1, D1), lambda s: (0, 0)),
            pl.BlockSpec((1, D2), lambda s: (0, 0)),
        ],
        out_specs=pl.BlockSpec((1, Q, D2), lambda s: (s, 0, 0)),
        out_shape=jax.ShapeDtypeStruct((S, Q, D2), jnp.float32),
        scratch_shapes=[pltpu.VMEM((NP, D1), jnp.float32)],
    )(a, qe_pad, w1, w2, b1, b2)


def kernel(slice_matrices, qubit_embs, W1, b1, W2, b2):
    sm = slice_matrices.astype(jnp.int32)
    a = _build_adjacency(sm)
    qe_pad = jnp.zeros((NP, D0), jnp.float32).at[:Q].set(qubit_embs)
    out = _gcn_tc(a, qe_pad, W1, W2, b1.reshape(1, D1), b2.reshape(1, D2))
    return out.reshape(S * Q, D2)


# speedup vs baseline: 81.2020x; 1.1094x over previous
import functools

import jax
import jax.numpy as jnp
from jax import lax
from jax.experimental import pallas as pl
from jax.experimental.pallas import tpu as pltpu
from jax.experimental.pallas import tpu_sc as plsc

S, Q, E, D0, D1, D2 = 20, 500, 16000, 128, 128, 128
NP = 512
QUARTER = 128
NTASK = S * (NP // QUARTER)
NWORKER = 32
CHUNK = 3200
NCHUNK = E // CHUNK
GROUPS = CHUNK // 16

_sc_mesh = plsc.VectorSubcoreMesh(core_axis_name="c", subcore_axis_name="s")


@functools.partial(
    pl.kernel,
    out_type=jax.ShapeDtypeStruct((S, NP, NP), jnp.float32),
    mesh=_sc_mesh,
    scratch_types=[
        pltpu.VMEM((2, 2, CHUNK), jnp.int32),
        pltpu.VMEM((QUARTER, NP), jnp.float32),
        pltpu.SemaphoreType.DMA,
        pltpu.SemaphoreType.DMA,
    ],
    compiler_params=pltpu.CompilerParams(needs_layout_passes=False),
)
def _build_adjacency(sm_hbm, a_hbm, ebuf_v, acc_v, sem0, sem1):
    wid = lax.axis_index("s") * 2 + lax.axis_index("c")
    zeros16 = jnp.zeros((16,), jnp.float32)
    ones16 = jnp.ones((16,), jnp.float32)
    sems = (sem0, sem1)
    UNROLL = 5

    def start_chunk(sl, c, buf):
        off = sl * (2 * E) + c * CHUNK
        d0 = pltpu.async_copy(sm_hbm.at[pl.ds(off, CHUNK)],
                              ebuf_v.at[buf, 0], sems[buf])
        d1 = pltpu.async_copy(sm_hbm.at[pl.ds(off + E, CHUNK)],
                              ebuf_v.at[buf, 1], sems[buf])
        return (d0, d1)

    for rnd in range((NTASK + NWORKER - 1) // NWORKER):
        task = rnd * NWORKER + wid

        @pl.when(task < NTASK)
        def _():
            sl = task % S
            base = (task // S) * QUARTER

            copies = [None] * NCHUNK
            copies[0] = start_chunk(sl, 0, 0)

            def zero_row(i, _):
                for j in range(NP // 16):
                    acc_v[i, pl.ds(j * 16, 16)] = zeros16
                return 0

            lax.fori_loop(0, QUARTER, zero_row, 0)

            for c in range(NCHUNK):
                cur = c % 2
                if c + 1 < NCHUNK:
                    copies[c + 1] = start_chunk(sl, c + 1, 1 - cur)
                copies[c][0].wait()
                copies[c][1].wait()

                def edge_group(g, _):
                    for u in range(UNROLL):
                        off = g * (16 * UNROLL) + u * 16
                        r16 = ebuf_v[cur, 0, pl.ds(off, 16)]
                        c16 = ebuf_v[cur, 1, pl.ds(off, 16)]
                        local = c16 - base
                        mask = (local >= 0) & (local < QUARTER)
                        local = jnp.where(mask, local, 0)
                        plsc.addupdate_scatter(
                            acc_v, [local, r16], ones16, mask=mask)
                    return 0

                lax.fori_loop(0, GROUPS // UNROLL, edge_group, 0)

            pltpu.sync_copy(acc_v, a_hbm.at[sl, pl.ds(base, QUARTER), :])


def _split(x):
    hi = x.astype(jnp.bfloat16)
    lo = (x - hi.astype(jnp.float32)).astype(jnp.bfloat16)
    return hi, lo


def _mm3(xs, ys):
    xh, xl = xs
    yh, yl = ys
    def d(p, q):
        return jnp.dot(p, q, preferred_element_type=jnp.float32)
    return d(xh, yh) + d(xh, yl) + d(xl, yh)


def _gcn_body(a_ref, qe_ref, w1_ref, w2_ref, b1_ref, b2_ref, out_ref, h0_ref):
    @pl.when(pl.program_id(0) == 0)
    def _():
        h0_ref[...] = _mm3(_split(qe_ref[...]), _split(w1_ref[...]))

    a = a_ref[0]
    deg = jnp.sum(a, axis=1) + 1.0
    dinv = lax.rsqrt(deg)
    i0 = lax.broadcasted_iota(jnp.int32, (NP, NP), 0)
    i1 = lax.broadcasted_iota(jnp.int32, (NP, NP), 1)
    eye = jnp.where(i0 == i1, 1.0, 0.0)
    m = (a + eye) * dinv[:, None] * dinv[None, :]
    ms = _split(m)
    w2s = _split(w2_ref[...])

    x1 = jnp.maximum(_mm3(ms, _split(h0_ref[...])) + b1_ref[...], 0.0)
    h2 = _mm3(_split(x1), w2s)
    x2 = jnp.maximum(_mm3(ms, _split(h2)) + b2_ref[...], 0.0)
    out_ref[0] = x2[:Q]


def _gcn_tc(a, qe_pad, w1, w2, b1, b2):
    return pl.pallas_call(
        _gcn_body,
        grid=(S,),
        in_specs=[
            pl.BlockSpec((1, NP, NP), lambda s: (s, 0, 0)),
            pl.BlockSpec((NP, D0), lambda s: (0, 0)),
            pl.BlockSpec((D0, D1), lambda s: (0, 0)),
            pl.BlockSpec((D1, D2), lambda s: (0, 0)),
            pl.BlockSpec((1, D1), lambda s: (0, 0)),
            pl.BlockSpec((1, D2), lambda s: (0, 0)),
        ],
        out_specs=pl.BlockSpec((1, Q, D2), lambda s: (s, 0, 0)),
        out_shape=jax.ShapeDtypeStruct((S, Q, D2), jnp.float32),
        scratch_shapes=[pltpu.VMEM((NP, D1), jnp.float32)],
    )(a, qe_pad, w1, w2, b1, b2)


def kernel(slice_matrices, qubit_embs, W1, b1, W2, b2):
    sm = slice_matrices.astype(jnp.int32).reshape(S * 2 * E)
    a = _build_adjacency(sm)
    qe_pad = jnp.zeros((NP, D0), jnp.float32).at[:Q].set(qubit_embs)
    out = _gcn_tc(a, qe_pad, W1, W2, b1.reshape(1, D1), b2.reshape(1, D2))
    return out.reshape(S * Q, D2)


# speedup vs baseline: 88.9702x; 1.0957x over previous
import functools

import jax
import jax.numpy as jnp
from jax import lax
from jax.experimental import pallas as pl
from jax.experimental.pallas import tpu as pltpu
from jax.experimental.pallas import tpu_sc as plsc

S, Q, E, D0, D1, D2 = 20, 500, 16000, 128, 128, 128
NP = 512
QUARTER = 128
NTASK = S * (NP // QUARTER)
NWORKER = 32
CHUNK = 3200
NCHUNK = E // CHUNK
GROUPS = CHUNK // 16

_sc_mesh = plsc.VectorSubcoreMesh(core_axis_name="c", subcore_axis_name="s")


@functools.partial(
    pl.kernel,
    out_type=jax.ShapeDtypeStruct((S, NP, NP), jnp.float32),
    mesh=_sc_mesh,
    scratch_types=[
        pltpu.VMEM((2, 2, CHUNK), jnp.int32),
        pltpu.VMEM((QUARTER, NP), jnp.float32),
        pltpu.SemaphoreType.DMA,
        pltpu.SemaphoreType.DMA,
    ],
    compiler_params=pltpu.CompilerParams(needs_layout_passes=False),
)
def _build_adjacency(sm_hbm, a_hbm, ebuf_v, acc_v, sem0, sem1):
    wid = lax.axis_index("s") * 2 + lax.axis_index("c")
    zeros16 = jnp.zeros((16,), jnp.float32)
    ones16 = jnp.ones((16,), jnp.float32)
    sems = (sem0, sem1)
    UNROLL = 5

    def start_chunk(sl, c, buf):
        return pltpu.async_copy(
            sm_hbm.at[sl, :, pl.ds(c * CHUNK, CHUNK)],
            ebuf_v.at[buf], sems[buf])

    for rnd in range((NTASK + NWORKER - 1) // NWORKER):
        task = rnd * NWORKER + wid

        @pl.when(task < NTASK)
        def _():
            sl = task % S
            base = (task // S) * QUARTER

            copies = [None] * NCHUNK
            copies[0] = start_chunk(sl, 0, 0)

            def zero_row(i, _):
                for j in range(NP // 16):
                    acc_v[i, pl.ds(j * 16, 16)] = zeros16
                return 0

            lax.fori_loop(0, QUARTER, zero_row, 0)

            for c in range(NCHUNK):
                cur = c % 2
                if c + 1 < NCHUNK:
                    copies[c + 1] = start_chunk(sl, c + 1, 1 - cur)
                copies[c].wait()

                def edge_group(g, _):
                    for u in range(UNROLL):
                        off = g * (16 * UNROLL) + u * 16
                        r16 = ebuf_v[cur, 0, pl.ds(off, 16)]
                        c16 = ebuf_v[cur, 1, pl.ds(off, 16)]
                        local = c16 - base
                        mask = local.astype(jnp.uint32) < QUARTER
                        plsc.addupdate_scatter(
                            acc_v, [local, r16], ones16, mask=mask)
                    return 0

                lax.fori_loop(0, GROUPS // UNROLL, edge_group, 0)

            pltpu.sync_copy(acc_v, a_hbm.at[sl, pl.ds(base, QUARTER), :])


def _split(x):
    hi = x.astype(jnp.bfloat16)
    lo = (x - hi.astype(jnp.float32)).astype(jnp.bfloat16)
    return hi, lo


def _mm3(xs, ys):
    xh, xl = xs
    yh, yl = ys
    def d(p, q):
        return jnp.dot(p, q, preferred_element_type=jnp.float32)
    return d(xh, yh) + d(xh, yl) + d(xl, yh)


def _gcn_body(a_ref, qe_ref, w1_ref, w2_ref, b1_ref, b2_ref, out_ref, h0_ref):
    @pl.when(pl.program_id(0) == 0)
    def _():
        h0_ref[...] = _mm3(_split(qe_ref[...]), _split(w1_ref[...]))

    a = a_ref[0]
    deg = jnp.sum(a, axis=1) + 1.0
    dinv = lax.rsqrt(deg)[:, None]
    asp = _split(a)
    w2s = _split(w2_ref[...])

    xs1 = dinv * h0_ref[...]
    x1 = jnp.maximum(dinv * (_mm3(asp, _split(xs1)) + xs1) + b1_ref[...], 0.0)
    xs2 = dinv * _mm3(_split(x1), w2s)
    x2 = jnp.maximum(dinv * (_mm3(asp, _split(xs2)) + xs2) + b2_ref[...], 0.0)
    out_ref[0] = x2[:Q]


def _gcn_tc(a, qe_pad, w1, w2, b1, b2):
    return pl.pallas_call(
        _gcn_body,
        grid=(S,),
        in_specs=[
            pl.BlockSpec((1, NP, NP), lambda s: (s, 0, 0)),
            pl.BlockSpec((NP, D0), lambda s: (0, 0)),
            pl.BlockSpec((D0, D1), lambda s: (0, 0)),
            pl.BlockSpec((D1, D2), lambda s: (0, 0)),
            pl.BlockSpec((1, D1), lambda s: (0, 0)),
            pl.BlockSpec((1, D2), lambda s: (0, 0)),
        ],
        out_specs=pl.BlockSpec((1, Q, D2), lambda s: (s, 0, 0)),
        out_shape=jax.ShapeDtypeStruct((S, Q, D2), jnp.float32),
        scratch_shapes=[pltpu.VMEM((NP, D1), jnp.float32)],
    )(a, qe_pad, w1, w2, b1, b2)


def kernel(slice_matrices, qubit_embs, W1, b1, W2, b2):
    sm = slice_matrices.astype(jnp.int32)
    a = _build_adjacency(sm)
    qe_pad = jnp.zeros((NP, D0), jnp.float32).at[:Q].set(qubit_embs)
    out = _gcn_tc(a, qe_pad, W1, W2, b1.reshape(1, D1), b2.reshape(1, D2))
    return out.reshape(S * Q, D2)


# speedup vs baseline: 89.0978x; 1.0014x over previous
import functools

import jax
import jax.numpy as jnp
from jax import lax
from jax.experimental import pallas as pl
from jax.experimental.pallas import tpu as pltpu
from jax.experimental.pallas import tpu_sc as plsc

S, Q, E, D0, D1, D2 = 20, 500, 16000, 128, 128, 128
NP = 512
QUARTER = 128
NTASK = S * (NP // QUARTER)
NWORKER = 32
CHUNK = 3200
NCHUNK = E // CHUNK
GROUPS = CHUNK // 16

_sc_mesh = plsc.VectorSubcoreMesh(core_axis_name="c", subcore_axis_name="s")


@functools.partial(
    pl.kernel,
    out_type=jax.ShapeDtypeStruct((S, NP, NP), jnp.float32),
    mesh=_sc_mesh,
    scratch_types=[
        pltpu.VMEM((2, 2, CHUNK), jnp.int32),
        pltpu.VMEM((QUARTER, NP), jnp.float32),
        pltpu.SemaphoreType.DMA,
        pltpu.SemaphoreType.DMA,
    ],
    compiler_params=pltpu.CompilerParams(needs_layout_passes=False),
)
def _build_adjacency(sm_hbm, a_hbm, ebuf_v, acc_v, sem0, sem1):
    wid = lax.axis_index("s") * 2 + lax.axis_index("c")
    zeros16 = jnp.zeros((16,), jnp.float32)
    ones16 = jnp.ones((16,), jnp.float32)
    sems = (sem0, sem1)
    UNROLL = 5

    def start_chunk(sl, c, buf):
        return pltpu.async_copy(
            sm_hbm.at[sl, :, pl.ds(c * CHUNK, CHUNK)],
            ebuf_v.at[buf], sems[buf])

    for rnd in range((NTASK + NWORKER - 1) // NWORKER):
        task = rnd * NWORKER + wid

        @pl.when(task < NTASK)
        def _():
            sl = task % S
            base = (task // S) * QUARTER

            copies = [None] * NCHUNK
            copies[0] = start_chunk(sl, 0, 0)

            def zero_row(i, _):
                for j in range(NP // 16):
                    acc_v[i, pl.ds(j * 16, 16)] = zeros16
                return 0

            lax.fori_loop(0, QUARTER, zero_row, 0)

            for c in range(NCHUNK):
                cur = c % 2
                if c + 1 < NCHUNK:
                    copies[c + 1] = start_chunk(sl, c + 1, 1 - cur)
                copies[c].wait()

                def edge_group(g, _):
                    for u in range(UNROLL):
                        off = g * (16 * UNROLL) + u * 16
                        r16 = ebuf_v[cur, 0, pl.ds(off, 16)]
                        c16 = ebuf_v[cur, 1, pl.ds(off, 16)]
                        local = c16 - base
                        mask = local.astype(jnp.uint32) < QUARTER
                        plsc.addupdate_scatter(
                            acc_v, [local, r16], ones16, mask=mask)
                    return 0

                lax.fori_loop(0, GROUPS // UNROLL, edge_group, 0)

            pltpu.sync_copy(acc_v, a_hbm.at[sl, pl.ds(base, QUARTER), :])


def _split(x):
    hi = x.astype(jnp.bfloat16)
    lo = (x - hi.astype(jnp.float32)).astype(jnp.bfloat16)
    return hi, lo


def _mm3(xs, ys):
    xh, xl = xs
    yh, yl = ys
    def d(p, q):
        return jnp.dot(p, q, preferred_element_type=jnp.float32)
    return d(xh, yh) + d(xh, yl) + d(xl, yh)


def _gcn_body(a_ref, qe_ref, w1_ref, w2_ref, b1_ref, b2_ref, out_ref, h0_ref):
    @pl.when(pl.program_id(0) == 0)
    def _():
        h0_ref[pl.ds(Q, NP - Q), :] = jnp.zeros((NP - Q, D1), jnp.float32)
        h0_ref[pl.ds(0, Q), :] = _mm3(_split(qe_ref[...]), _split(w1_ref[...]))

    a = a_ref[0]
    deg = jnp.sum(a, axis=1) + 1.0
    dinv = lax.rsqrt(deg)[:, None]
    asp = _split(a)
    w2s = _split(w2_ref[...])

    xs1 = dinv * h0_ref[...]
    x1 = jnp.maximum(dinv * (_mm3(asp, _split(xs1)) + xs1) + b1_ref[...], 0.0)
    xs2 = dinv * _mm3(_split(x1), w2s)
    x2 = jnp.maximum(dinv * (_mm3(asp, _split(xs2)) + xs2) + b2_ref[...], 0.0)
    out_ref[0] = x2[:Q]


def _gcn_tc(a, qe_pad, w1, w2, b1, b2):
    return pl.pallas_call(
        _gcn_body,
        grid=(S,),
        in_specs=[
            pl.BlockSpec((1, NP, NP), lambda s: (s, 0, 0)),
            pl.BlockSpec((Q, D0), lambda s: (0, 0)),
            pl.BlockSpec((D0, D1), lambda s: (0, 0)),
            pl.BlockSpec((D1, D2), lambda s: (0, 0)),
            pl.BlockSpec((D1,), lambda s: (0,)),
            pl.BlockSpec((D2,), lambda s: (0,)),
        ],
        out_specs=pl.BlockSpec((1, Q, D2), lambda s: (s, 0, 0)),
        out_shape=jax.ShapeDtypeStruct((S, Q, D2), jnp.float32),
        scratch_shapes=[pltpu.VMEM((NP, D1), jnp.float32)],
    )(a, qe_pad, w1, w2, b1, b2)


def kernel(slice_matrices, qubit_embs, W1, b1, W2, b2):
    sm = slice_matrices.astype(jnp.int32)
    a = _build_adjacency(sm)
    out = _gcn_tc(a, qubit_embs, W1, W2, b1, b2)
    return out.reshape(S * Q, D2)
